# Initial kernel scaffold; baseline (speedup 1.0000x reference)
import functools

import jax
import jax.numpy as jnp
from jax import lax
from jax.experimental import pallas as pl
from jax.experimental.pallas import tpu as pltpu
from jax.experimental.pallas import tpu_sc as plsc

N_ROWS = 65536
N_COLS = 65536
NNZ = 4194304

NW = 32
ROWS_W = N_ROWS // NW
CHUNK = 4096
NG = CHUNK // 16
RG = ROWS_W // 16

_mesh = plsc.VectorSubcoreMesh(core_axis_name="c", subcore_axis_name="s")


@functools.partial(
    pl.kernel,
    out_type=jax.ShapeDtypeStruct((N_ROWS,), jnp.float32),
    mesh=_mesh,
    scratch_types=[
        pltpu.VMEM((N_COLS,), jnp.float32),
        pltpu.VMEM((ROWS_W + 8,), jnp.int32),
        pltpu.VMEM((CHUNK,), jnp.float32),
        pltpu.VMEM((CHUNK,), jnp.int32),
        pltpu.VMEM((CHUNK,), jnp.float32),
        pltpu.VMEM((ROWS_W + 16,), jnp.float32),
        pltpu.VMEM((ROWS_W,), jnp.float32),
    ],
)
def _spmv_sc(sx_hbm, x_hbm, idx_hbm, ro_hbm, o_hbm,
             x_v, ro_v, sx_v, ix_v, s_v, e_v, y_v):
    wid = lax.axis_index("c") * 16 + lax.axis_index("s")
    r0 = wid * ROWS_W
    i16 = lax.iota(jnp.int32, 16)

    pltpu.sync_copy(x_hbm, x_v)
    pltpu.sync_copy(ro_hbm.at[pl.ds(r0, ROWS_W + 8)], ro_v)
    s0 = ro_v[0]
    s1 = ro_v[ROWS_W]

    z16 = jnp.zeros((16,), jnp.float32)

    @pl.loop(0, ROWS_W + 16, step=16)
    def _(j):
        e_v[pl.ds(j, 16)] = z16

    k0 = s0 // CHUNK
    k1 = (s1 + CHUNK - 1) // CHUNK

    def chunk_body(k, st):
        gcur, carry = st
        a = k * CHUNK
        pltpu.sync_copy(sx_hbm.at[pl.ds(a, CHUNK)], sx_v)
        pltpu.sync_copy(idx_hbm.at[pl.ds(a, CHUNK)], ix_v)

        def grp(j, c):
            o = j * 16
            iv = ix_v[pl.ds(o, 16)]
            sv = sx_v[pl.ds(o, 16)]
            g = plsc.load_gather(x_v, [iv])
            pos = (a + o) + i16
            m = (pos >= s0) & (pos < s1)
            p = jnp.where(m, sv * g, 0.0)
            s_v[pl.ds(o, 16)] = plsc.cumsum(p) + c
            return c + jnp.sum(p)

        carry = lax.fori_loop(0, NG, grp, carry)

        cend = a + CHUNK

        def ext_cond(st2):
            g, stop = st2
            return jnp.logical_and(jnp.logical_not(stop), g < RG)

        def ext_body(st2):
            g, _ = st2
            tfirst = ro_v[1 + g * 16]
            do = tfirst <= cend

            @pl.when(do)
            def _():
                t = plsc.load_gather(ro_v, [i16 + (1 + g * 16)])
                m = (t > a) & (t <= cend)
                gi = jnp.clip(t - 1 - a, 0, CHUNK - 1)
                pv = plsc.load_gather(s_v, [gi])
                eo = e_v[pl.ds(16 + g * 16, 16)]
                e_v[pl.ds(16 + g * 16, 16)] = jnp.where(m, pv, eo)

            return (jnp.where(do, g + 1, g), jnp.logical_not(do))

        gend, _ = lax.while_loop(ext_cond, ext_body,
                                 (gcur, jnp.asarray(False)))
        return (jnp.maximum(gcur, gend - 1), carry)

    lax.fori_loop(k0, k1, chunk_body, (jnp.int32(0), jnp.float32(0.0)))

    @pl.loop(0, ROWS_W, step=16)
    def _(j):
        av = e_v[pl.ds(16 + j, 16)]
        bv = plsc.load_gather(e_v, [i16 + (15 + j)])
        y_v[pl.ds(j, 16)] = av - bv

    pltpu.sync_copy(y_v, o_hbm.at[pl.ds(r0, ROWS_W)])


def kernel(sx, x, y, selector_idx, row_end_offsets):
    del y
    ro_pad = jnp.concatenate(
        [row_end_offsets, jnp.broadcast_to(row_end_offsets[-1:], (7,))])
    return _spmv_sc(sx, x, selector_idx, ro_pad)


# speedup vs baseline: 3518.3622x; 3518.3622x over previous
import dataclasses
import functools

import jax
import jax.numpy as jnp
from jax import lax
from jax.experimental import pallas as pl
from jax.experimental.pallas import tpu as pltpu
from jax.experimental.pallas import tpu_sc as plsc

N_ROWS = 65536
N_COLS = 65536
NNZ = 4194304

NW = 32
ROWS_W = N_ROWS // NW
CHUNK = 4096
NG = CHUNK // 16
RG = ROWS_W // 16

_mesh = plsc.VectorSubcoreMesh(core_axis_name="c", subcore_axis_name="s")

_cp = pltpu.CompilerParams()
if "needs_layout_passes" in pltpu.CompilerParams.__dataclass_fields__:
    _cp = dataclasses.replace(_cp, needs_layout_passes=False)


@functools.partial(
    pl.kernel,
    out_type=jax.ShapeDtypeStruct((N_ROWS,), jnp.float32),
    mesh=_mesh,
    compiler_params=_cp,
    scratch_types=[
        pltpu.VMEM((N_COLS,), jnp.float32),
        pltpu.VMEM((ROWS_W + 16,), jnp.int32),
        pltpu.VMEM((CHUNK,), jnp.float32),
        pltpu.VMEM((CHUNK,), jnp.int32),
        pltpu.VMEM((CHUNK,), jnp.float32),
        pltpu.VMEM((ROWS_W + 16,), jnp.float32),
        pltpu.VMEM((ROWS_W,), jnp.float32),
    ],
)
def _spmv_sc(sx_hbm, x_hbm, idx_hbm, ro_hbm, o_hbm,
             x_v, ro_v, sx_v, ix_v, s_v, e_v, y_v):
    wid = lax.axis_index("c") * 16 + lax.axis_index("s")
    r0 = wid * ROWS_W
    i16 = lax.iota(jnp.int32, 16)

    pltpu.sync_copy(x_hbm, x_v)
    pltpu.sync_copy(ro_hbm.at[pl.ds(r0, ROWS_W + 8)],
                    ro_v.at[pl.ds(0, ROWS_W + 8)])
    s0 = ro_v[pl.ds(0, 16)][0]
    s1 = ro_v[pl.ds(ROWS_W, 16)][0]

    z16 = jnp.zeros((16,), jnp.float32)

    @pl.loop(0, ROWS_W + 16, step=16)
    def _(j):
        e_v[pl.ds(j, 16)] = z16

    k0 = s0 // CHUNK
    k1 = (s1 + CHUNK - 1) // CHUNK

    def chunk_body(k, st):
        gcur, carry = st
        a = k * CHUNK
        pltpu.sync_copy(sx_hbm.at[pl.ds(a, CHUNK)], sx_v)
        pltpu.sync_copy(idx_hbm.at[pl.ds(a, CHUNK)], ix_v)

        def grp(j, c):
            o = j * 16
            iv = ix_v[pl.ds(o, 16)]
            sv = sx_v[pl.ds(o, 16)]
            g = plsc.load_gather(x_v, [iv])
            pos = (a + o) + i16
            m = (pos >= s0) & (pos < s1)
            p = jnp.where(m, sv * g, 0.0)
            s_v[pl.ds(o, 16)] = plsc.cumsum(p) + c
            return c + jnp.sum(p)

        carry = lax.fori_loop(0, NG, grp, carry)

        cend = a + CHUNK

        def ext_cond(st2):
            g, stop = st2
            return jnp.logical_and(jnp.logical_not(stop), g < RG)

        def ext_body(st2):
            g, _ = st2
            tfirst = ro_v[pl.ds(g * 16, 16)][1]
            do = tfirst <= cend

            @pl.when(do)
            def _():
                t = plsc.load_gather(ro_v, [i16 + (1 + g * 16)])
                m = (t > a) & (t <= cend)
                gi = jnp.clip(t - 1 - a, 0, CHUNK - 1)
                pv = plsc.load_gather(s_v, [gi])
                eo = e_v[pl.ds(16 + g * 16, 16)]
                e_v[pl.ds(16 + g * 16, 16)] = jnp.where(m, pv, eo)

            return (jnp.where(do, g + 1, g), jnp.logical_not(do))

        gend, _ = lax.while_loop(ext_cond, ext_body,
                                 (gcur, jnp.asarray(False)))
        return (jnp.maximum(gcur, gend - 1), carry)

    lax.fori_loop(k0, k1, chunk_body, (jnp.int32(0), jnp.float32(0.0)))

    @pl.loop(0, ROWS_W, step=16)
    def _(j):
        av = e_v[pl.ds(16 + j, 16)]
        bv = plsc.load_gather(e_v, [i16 + (15 + j)])
        y_v[pl.ds(j, 16)] = av - bv

    pltpu.sync_copy(y_v, o_hbm.at[pl.ds(r0, ROWS_W)])


def kernel(sx, x, y, selector_idx, row_end_offsets):
    del y
    ro_pad = jnp.concatenate(
        [row_end_offsets, jnp.broadcast_to(row_end_offsets[-1:], (7,))])
    return _spmv_sc(sx, x, selector_idx, ro_pad)


# speedup vs baseline: 3718.8699x; 1.0570x over previous
import dataclasses
import functools

import jax
import jax.numpy as jnp
from jax import lax
from jax.experimental import pallas as pl
from jax.experimental.pallas import tpu as pltpu
from jax.experimental.pallas import tpu_sc as plsc

N_ROWS = 65536
N_COLS = 65536
NNZ = 4194304

NW = 32
ROWS_W = N_ROWS // NW
CHUNK = 4096
NG = CHUNK // 16
RG = ROWS_W // 16

_mesh = plsc.VectorSubcoreMesh(core_axis_name="c", subcore_axis_name="s")

_cp = pltpu.CompilerParams()
if "needs_layout_passes" in pltpu.CompilerParams.__dataclass_fields__:
    _cp = dataclasses.replace(_cp, needs_layout_passes=False)


@functools.partial(
    pl.kernel,
    out_type=jax.ShapeDtypeStruct((N_ROWS,), jnp.float32),
    mesh=_mesh,
    compiler_params=_cp,
    scratch_types=[
        pltpu.VMEM((N_COLS,), jnp.float32),
        pltpu.VMEM((ROWS_W + 16,), jnp.int32),
        pltpu.VMEM((CHUNK,), jnp.float32),
        pltpu.VMEM((CHUNK,), jnp.int32),
        pltpu.VMEM((CHUNK,), jnp.float32),
        pltpu.VMEM((CHUNK,), jnp.int32),
        pltpu.VMEM((CHUNK,), jnp.float32),
        pltpu.VMEM((ROWS_W + 16,), jnp.float32),
        pltpu.VMEM((ROWS_W,), jnp.float32),
        pltpu.SemaphoreType.DMA,
        pltpu.SemaphoreType.DMA,
    ],
)
def _spmv_sc(sx_hbm, x_hbm, idx_hbm, ro_hbm, o_hbm,
             x_v, ro_v, sx_v0, ix_v0, sx_v1, ix_v1, s_v, e_v, y_v,
             sem0, sem1):
    wid = lax.axis_index("c") * 16 + lax.axis_index("s")
    r0 = wid * ROWS_W
    i16 = lax.iota(jnp.int32, 16)

    pltpu.sync_copy(x_hbm, x_v)
    pltpu.sync_copy(ro_hbm.at[pl.ds(r0, ROWS_W + 8)],
                    ro_v.at[pl.ds(0, ROWS_W + 8)])
    s0 = ro_v[pl.ds(0, 16)][0]
    s1 = ro_v[pl.ds(ROWS_W, 16)][0]

    z16 = jnp.zeros((16,), jnp.float32)

    @pl.loop(0, ROWS_W + 16, step=16)
    def _(j):
        e_v[pl.ds(j, 16)] = z16

    k0 = s0 // CHUNK
    k1 = (s1 + CHUNK - 1) // CHUNK

    def start_chunk(k, sxb, ixb, sem):
        @pl.when(k < k1)
        def _():
            a = k * CHUNK
            pltpu.async_copy(sx_hbm.at[pl.ds(a, CHUNK)], sxb, sem)
            pltpu.async_copy(idx_hbm.at[pl.ds(a, CHUNK)], ixb, sem)

    def wait_chunk(do, sxb, ixb, sem):
        @pl.when(do)
        def _():
            pltpu.make_async_copy(sx_hbm.at[pl.ds(0, CHUNK)], sxb, sem).wait()
            pltpu.make_async_copy(idx_hbm.at[pl.ds(0, CHUNK)], ixb, sem).wait()

    def compute_chunk(k, gcur, carry, sxb, ixb):
        a = k * CHUNK

        def grp(j, c):
            o = j * 16
            iv = ixb[pl.ds(o, 16)]
            sv = sxb[pl.ds(o, 16)]
            g = plsc.load_gather(x_v, [iv])
            pos = (a + o) + i16
            m = (pos >= s0) & (pos < s1)
            p = jnp.where(m, sv * g, 0.0)
            s = plsc.cumsum(p) + c
            s_v[pl.ds(o, 16)] = s
            return s[15]

        carry = lax.fori_loop(0, NG, grp, carry, unroll=4)

        cend = a + CHUNK

        def ext_cond(st2):
            g, stop = st2
            return jnp.logical_and(jnp.logical_not(stop), g < RG)

        def ext_body(st2):
            g, _ = st2
            tfirst = ro_v[pl.ds(g * 16, 16)][1]
            do = tfirst <= cend

            @pl.when(do)
            def _():
                t = plsc.load_gather(ro_v, [i16 + (1 + g * 16)])
                m = (t > a) & (t <= cend)
                gi = jnp.clip(t - 1 - a, 0, CHUNK - 1)
                pv = plsc.load_gather(s_v, [gi])
                eo = e_v[pl.ds(16 + g * 16, 16)]
                e_v[pl.ds(16 + g * 16, 16)] = jnp.where(m, pv, eo)

            return (jnp.where(do, g + 1, g), jnp.logical_not(do))

        gend, _ = lax.while_loop(ext_cond, ext_body,
                                 (gcur, jnp.asarray(False)))
        return jnp.maximum(gcur, gend - 1), carry

    def guarded_compute(k, gcur, carry, sxb, ixb):
        return lax.cond(
            k < k1,
            lambda: compute_chunk(k, gcur, carry, sxb, ixb),
            lambda: (gcur, carry),
        )

    start_chunk(k0, sx_v0, ix_v0, sem0)
    start_chunk(k0 + 1, sx_v1, ix_v1, sem1)

    def pair_body(i, st):
        gcur, carry = st
        ka = k0 + 2 * i
        kb = ka + 1
        wait_chunk(ka < k1, sx_v0, ix_v0, sem0)
        gcur, carry = guarded_compute(ka, gcur, carry, sx_v0, ix_v0)
        start_chunk(ka + 2, sx_v0, ix_v0, sem0)
        wait_chunk(kb < k1, sx_v1, ix_v1, sem1)
        gcur, carry = guarded_compute(kb, gcur, carry, sx_v1, ix_v1)
        start_chunk(kb + 2, sx_v1, ix_v1, sem1)
        return (gcur, carry)

    npairs = (k1 - k0 + 1) // 2
    lax.fori_loop(0, npairs, pair_body, (jnp.int32(0), jnp.float32(0.0)))

    @pl.loop(0, ROWS_W, step=16)
    def _(j):
        av = e_v[pl.ds(16 + j, 16)]
        bv = plsc.load_gather(e_v, [i16 + (15 + j)])
        y_v[pl.ds(j, 16)] = av - bv

    pltpu.sync_copy(y_v, o_hbm.at[pl.ds(r0, ROWS_W)])


def kernel(sx, x, y, selector_idx, row_end_offsets):
    del y
    ro_pad = jnp.concatenate(
        [row_end_offsets, jnp.broadcast_to(row_end_offsets[-1:], (7,))])
    return _spmv_sc(sx, x, selector_idx, ro_pad)


# speedup vs baseline: 11041.2307x; 2.9690x over previous
import dataclasses
import functools

import jax
import jax.numpy as jnp
from jax import lax
from jax.experimental import pallas as pl
from jax.experimental.pallas import tpu as pltpu
from jax.experimental.pallas import tpu_sc as plsc

N_ROWS = 65536
N_COLS = 65536
NNZ = 4194304

NW = 32
ROWS_W = N_ROWS // NW
CHUNK = 4096
NG = CHUNK // 16
RG = ROWS_W // 16

_mesh = plsc.VectorSubcoreMesh(core_axis_name="c", subcore_axis_name="s")

_cp = pltpu.CompilerParams()
if "needs_layout_passes" in pltpu.CompilerParams.__dataclass_fields__:
    _cp = dataclasses.replace(_cp, needs_layout_passes=False)


@functools.partial(
    pl.kernel,
    out_type=jax.ShapeDtypeStruct((N_ROWS,), jnp.float32),
    mesh=_mesh,
    compiler_params=_cp,
    scratch_types=[
        pltpu.VMEM((N_COLS,), jnp.float32),
        pltpu.VMEM((ROWS_W + 16,), jnp.int32),
        pltpu.VMEM((CHUNK,), jnp.float32),
        pltpu.VMEM((CHUNK,), jnp.int32),
        pltpu.VMEM((CHUNK,), jnp.float32),
        pltpu.VMEM((CHUNK,), jnp.int32),
        pltpu.VMEM((CHUNK,), jnp.float32),
        pltpu.VMEM((NG,), jnp.float32),
        pltpu.VMEM((ROWS_W + 16,), jnp.float32),
        pltpu.VMEM((ROWS_W,), jnp.float32),
        pltpu.SemaphoreType.DMA,
        pltpu.SemaphoreType.DMA,
    ],
)
def _spmv_sc(sx_hbm, x_hbm, idx_hbm, ro_hbm, o_hbm,
             x_v, ro_v, sx_v0, ix_v0, sx_v1, ix_v1, s_v, gs_v, e_v, y_v,
             sem0, sem1):
    wid = lax.axis_index("c") * 16 + lax.axis_index("s")
    r0 = wid * ROWS_W
    i16 = lax.iota(jnp.int32, 16)

    pltpu.sync_copy(x_hbm, x_v)
    pltpu.sync_copy(ro_hbm.at[pl.ds(r0, ROWS_W + 8)],
                    ro_v.at[pl.ds(0, ROWS_W + 8)])
    s0 = ro_v[pl.ds(0, 16)][0]
    s1 = ro_v[pl.ds(ROWS_W, 16)][0]

    z16 = jnp.zeros((16,), jnp.float32)

    @pl.loop(0, ROWS_W + 16, step=16)
    def _(j):
        e_v[pl.ds(j, 16)] = z16

    k0 = s0 // CHUNK
    k1 = (s1 + CHUNK - 1) // CHUNK

    def start_chunk(k, sxb, ixb, sem):
        @pl.when(k < k1)
        def _():
            a = k * CHUNK
            pltpu.async_copy(sx_hbm.at[pl.ds(a, CHUNK)], sxb, sem)
            pltpu.async_copy(idx_hbm.at[pl.ds(a, CHUNK)], ixb, sem)

    def wait_chunk(do, sxb, ixb, sem):
        @pl.when(do)
        def _():
            pltpu.make_async_copy(sx_hbm.at[pl.ds(0, CHUNK)], sxb, sem).wait()
            pltpu.make_async_copy(idx_hbm.at[pl.ds(0, CHUNK)], ixb, sem).wait()

    def compute_chunk(k, gcur, carry, sxb, ixb):
        a = k * CHUNK

        @plsc.parallel_loop(0, NG, unroll=8)
        def _(j):
            o = j * 16
            iv = ixb[pl.ds(o, 16)]
            sv = sxb[pl.ds(o, 16)]
            g = plsc.load_gather(x_v, [iv])
            pos = (a + o) + i16
            m = (pos >= s0) & (pos < s1)
            p = jnp.where(m, sv * g, 0.0)
            s_v[pl.ds(o, 16)] = plsc.cumsum(p)

        def p2(jj, c2):
            tot = plsc.load_gather(s_v, [i16 * 16 + (jj * 256 + 15)])
            s2 = plsc.cumsum(tot) + c2
            gs_v[pl.ds(jj * 16, 16)] = s2 - tot
            return s2[15]

        carry = lax.fori_loop(0, NG // 16, p2, carry)

        cend = a + CHUNK

        def ext_cond(st2):
            g, stop = st2
            return jnp.logical_and(jnp.logical_not(stop), g < RG)

        def ext_body(st2):
            g, _ = st2
            tfirst = ro_v[pl.ds(g * 16, 16)][1]
            do = tfirst <= cend

            @pl.when(do)
            def _():
                t = plsc.load_gather(ro_v, [i16 + (1 + g * 16)])
                m = (t > a) & (t <= cend)
                gi = jnp.clip(t - 1 - a, 0, CHUNK - 1)
                pv = (plsc.load_gather(s_v, [gi])
                      + plsc.load_gather(gs_v, [lax.shift_right_logical(gi, 4)]))
                eo = e_v[pl.ds(16 + g * 16, 16)]
                e_v[pl.ds(16 + g * 16, 16)] = jnp.where(m, pv, eo)

            return (jnp.where(do, g + 1, g), jnp.logical_not(do))

        gend, _ = lax.while_loop(ext_cond, ext_body,
                                 (gcur, jnp.asarray(False)))
        return jnp.maximum(gcur, gend - 1), carry

    def guarded_compute(k, gcur, carry, sxb, ixb):
        return lax.cond(
            k < k1,
            lambda: compute_chunk(k, gcur, carry, sxb, ixb),
            lambda: (gcur, carry),
        )

    start_chunk(k0, sx_v0, ix_v0, sem0)
    start_chunk(k0 + 1, sx_v1, ix_v1, sem1)

    def pair_body(i, st):
        gcur, carry = st
        ka = k0 + 2 * i
        kb = ka + 1
        wait_chunk(ka < k1, sx_v0, ix_v0, sem0)
        gcur, carry = guarded_compute(ka, gcur, carry, sx_v0, ix_v0)
        start_chunk(ka + 2, sx_v0, ix_v0, sem0)
        wait_chunk(kb < k1, sx_v1, ix_v1, sem1)
        gcur, carry = guarded_compute(kb, gcur, carry, sx_v1, ix_v1)
        start_chunk(kb + 2, sx_v1, ix_v1, sem1)
        return (gcur, carry)

    npairs = (k1 - k0 + 1) // 2
    lax.fori_loop(0, npairs, pair_body, (jnp.int32(0), jnp.float32(0.0)))

    @pl.loop(0, ROWS_W, step=16)
    def _(j):
        av = e_v[pl.ds(16 + j, 16)]
        bv = plsc.load_gather(e_v, [i16 + (15 + j)])
        y_v[pl.ds(j, 16)] = av - bv

    pltpu.sync_copy(y_v, o_hbm.at[pl.ds(r0, ROWS_W)])


def kernel(sx, x, y, selector_idx, row_end_offsets):
    del y
    ro_pad = jnp.concatenate(
        [row_end_offsets, jnp.broadcast_to(row_end_offsets[-1:], (7,))])
    return _spmv_sc(sx, x, selector_idx, ro_pad)


# speedup vs baseline: 11813.0628x; 1.0699x over previous
import dataclasses
import functools

import jax
import jax.numpy as jnp
from jax import lax
from jax.experimental import pallas as pl
from jax.experimental.pallas import tpu as pltpu
from jax.experimental.pallas import tpu_sc as plsc

N_ROWS = 65536
N_COLS = 65536
NNZ = 4194304

NW = 32
ROWS_W = N_ROWS // NW
CHUNK = 8192
NG = CHUNK // 16
NB = NG // 16
RG = ROWS_W // 16

_mesh = plsc.VectorSubcoreMesh(core_axis_name="c", subcore_axis_name="s")

_cp = pltpu.CompilerParams()
if "needs_layout_passes" in pltpu.CompilerParams.__dataclass_fields__:
    _cp = dataclasses.replace(_cp, needs_layout_passes=False)


@functools.partial(
    pl.kernel,
    out_type=jax.ShapeDtypeStruct((N_ROWS,), jnp.float32),
    mesh=_mesh,
    compiler_params=_cp,
    scratch_types=[
        pltpu.VMEM((N_COLS,), jnp.float32),
        pltpu.VMEM((ROWS_W + 16,), jnp.int32),
        pltpu.VMEM((CHUNK,), jnp.float32),
        pltpu.VMEM((CHUNK,), jnp.int32),
        pltpu.VMEM((CHUNK,), jnp.float32),
        pltpu.VMEM((CHUNK,), jnp.int32),
        pltpu.VMEM((CHUNK,), jnp.float32),
        pltpu.VMEM((NG,), jnp.float32),
        pltpu.VMEM((NB,), jnp.float32),
        pltpu.VMEM((ROWS_W + 16,), jnp.float32),
        pltpu.VMEM((ROWS_W,), jnp.float32),
        pltpu.SemaphoreType.DMA,
        pltpu.SemaphoreType.DMA,
    ],
)
def _spmv_sc(sx_hbm, x_hbm, idx_hbm, ro_hbm, o_hbm,
             x_v, ro_v, sx_v0, ix_v0, sx_v1, ix_v1, s_v, l2_v, l3_v, e_v, y_v,
             sem0, sem1):
    wid = lax.axis_index("c") * 16 + lax.axis_index("s")
    r0 = wid * ROWS_W
    i16 = lax.iota(jnp.int32, 16)

    pltpu.sync_copy(x_hbm, x_v)
    pltpu.sync_copy(ro_hbm.at[pl.ds(r0, ROWS_W + 8)],
                    ro_v.at[pl.ds(0, ROWS_W + 8)])
    s0 = ro_v[pl.ds(0, 16)][0]
    s1 = ro_v[pl.ds(ROWS_W, 16)][0]

    z16 = jnp.zeros((16,), jnp.float32)

    @pl.loop(0, ROWS_W + 16, step=16)
    def _(j):
        e_v[pl.ds(j, 16)] = z16

    k0 = s0 // CHUNK
    k1 = (s1 + CHUNK - 1) // CHUNK

    def start_chunk(k, sxb, ixb, sem):
        @pl.when(k < k1)
        def _():
            a = k * CHUNK
            pltpu.async_copy(sx_hbm.at[pl.ds(a, CHUNK)], sxb, sem)
            pltpu.async_copy(idx_hbm.at[pl.ds(a, CHUNK)], ixb, sem)

    def wait_chunk(do, sxb, ixb, sem):
        @pl.when(do)
        def _():
            pltpu.make_async_copy(sx_hbm.at[pl.ds(0, CHUNK)], sxb, sem).wait()
            pltpu.make_async_copy(idx_hbm.at[pl.ds(0, CHUNK)], ixb, sem).wait()

    def compute_chunk(k, gcur, carry, sxb, ixb):
        a = k * CHUNK

        @plsc.parallel_loop(0, NG, unroll=8)
        def _(j):
            o = j * 16
            iv = ixb[pl.ds(o, 16)]
            sv = sxb[pl.ds(o, 16)]
            g = plsc.load_gather(x_v, [iv])
            pos = (a + o) + i16
            m = (pos >= s0) & (pos < s1)
            p = jnp.where(m, sv * g, 0.0)
            s_v[pl.ds(o, 16)] = plsc.cumsum(p)

        @plsc.parallel_loop(0, NB, unroll=2)
        def _(b):
            tot = plsc.load_gather(s_v, [i16 * 16 + (b * 256 + 15)])
            l2 = plsc.cumsum(tot)
            l2_v[pl.ds(b * 16, 16)] = l2 - tot

        def p2b(bb, c2):
            t_ex = plsc.load_gather(l2_v, [i16 * 16 + (bb * 256 + 15)])
            t_g15 = plsc.load_gather(s_v, [i16 * 256 + (bb * 4096 + 255)])
            btot = t_ex + t_g15
            l3 = plsc.cumsum(btot) + c2
            l3_v[pl.ds(bb * 16, 16)] = l3 - btot
            return l3[15]

        carry = lax.fori_loop(0, NB // 16, p2b, carry)

        cend = a + CHUNK

        def ext_cond(st2):
            g, stop = st2
            return jnp.logical_and(jnp.logical_not(stop), g < RG)

        def ext_body(st2):
            g, _ = st2
            tfirst = ro_v[pl.ds(g * 16, 16)][1]
            do = tfirst <= cend

            @pl.when(do)
            def _():
                t = plsc.load_gather(ro_v, [i16 + (1 + g * 16)])
                m = (t > a) & (t <= cend)
                gi = jnp.clip(t - 1 - a, 0, CHUNK - 1)
                pv = (plsc.load_gather(s_v, [gi])
                      + plsc.load_gather(l2_v, [lax.shift_right_logical(gi, 4)])
                      + plsc.load_gather(l3_v, [lax.shift_right_logical(gi, 8)]))
                eo = e_v[pl.ds(16 + g * 16, 16)]
                e_v[pl.ds(16 + g * 16, 16)] = jnp.where(m, pv, eo)

            return (jnp.where(do, g + 1, g), jnp.logical_not(do))

        gend, _ = lax.while_loop(ext_cond, ext_body,
                                 (gcur, jnp.asarray(False)))
        return jnp.maximum(gcur, gend - 1), carry

    def guarded_compute(k, gcur, carry, sxb, ixb):
        return lax.cond(
            k < k1,
            lambda: compute_chunk(k, gcur, carry, sxb, ixb),
            lambda: (gcur, carry),
        )

    start_chunk(k0, sx_v0, ix_v0, sem0)
    start_chunk(k0 + 1, sx_v1, ix_v1, sem1)

    def pair_body(i, st):
        gcur, carry = st
        ka = k0 + 2 * i
        kb = ka + 1
        wait_chunk(ka < k1, sx_v0, ix_v0, sem0)
        gcur, carry = guarded_compute(ka, gcur, carry, sx_v0, ix_v0)
        start_chunk(ka + 2, sx_v0, ix_v0, sem0)
        wait_chunk(kb < k1, sx_v1, ix_v1, sem1)
        gcur, carry = guarded_compute(kb, gcur, carry, sx_v1, ix_v1)
        start_chunk(kb + 2, sx_v1, ix_v1, sem1)
        return (gcur, carry)

    npairs = (k1 - k0 + 1) // 2
    lax.fori_loop(0, npairs, pair_body, (jnp.int32(0), jnp.float32(0.0)))

    @pl.loop(0, ROWS_W, step=16)
    def _(j):
        av = e_v[pl.ds(16 + j, 16)]
        bv = plsc.load_gather(e_v, [i16 + (15 + j)])
        y_v[pl.ds(j, 16)] = av - bv

    pltpu.sync_copy(y_v, o_hbm.at[pl.ds(r0, ROWS_W)])


def kernel(sx, x, y, selector_idx, row_end_offsets):
    del y
    ro_pad = jnp.concatenate(
        [row_end_offsets, jnp.broadcast_to(row_end_offsets[-1:], (7,))])
    return _spmv_sc(sx, x, selector_idx, ro_pad)


# speedup vs baseline: 11915.5944x; 1.0087x over previous
import dataclasses
import functools

import jax
import jax.numpy as jnp
from jax import lax
from jax.experimental import pallas as pl
from jax.experimental.pallas import tpu as pltpu
from jax.experimental.pallas import tpu_sc as plsc

N_ROWS = 65536
N_COLS = 65536
NNZ = 4194304

NW = 32
ROWS_W = N_ROWS // NW
CHUNK = 8192
NG = CHUNK // 16
NB = NG // 16
RG = ROWS_W // 16

_mesh = plsc.VectorSubcoreMesh(core_axis_name="c", subcore_axis_name="s")

_cp = pltpu.CompilerParams()
if "needs_layout_passes" in pltpu.CompilerParams.__dataclass_fields__:
    _cp = dataclasses.replace(_cp, needs_layout_passes=False)


@functools.partial(
    pl.kernel,
    out_type=jax.ShapeDtypeStruct((N_ROWS,), jnp.float32),
    mesh=_mesh,
    compiler_params=_cp,
    scratch_types=[
        pltpu.VMEM((N_COLS,), jnp.float32),
        pltpu.VMEM((ROWS_W + 16,), jnp.int32),
        pltpu.VMEM((CHUNK,), jnp.float32),
        pltpu.VMEM((CHUNK,), jnp.int32),
        pltpu.VMEM((CHUNK,), jnp.float32),
        pltpu.VMEM((CHUNK,), jnp.int32),
        pltpu.VMEM((CHUNK,), jnp.float32),
        pltpu.VMEM((NG,), jnp.float32),
        pltpu.VMEM((NB,), jnp.float32),
        pltpu.VMEM((ROWS_W + 16,), jnp.float32),
        pltpu.VMEM((ROWS_W,), jnp.float32),
        pltpu.SemaphoreType.DMA,
        pltpu.SemaphoreType.DMA,
    ],
)
def _spmv_sc(sx_hbm, x_hbm, idx_hbm, ro_hbm, o_hbm,
             x_v, ro_v, sx_v0, ix_v0, sx_v1, ix_v1, s_v, l2_v, l3_v, e_v, y_v,
             sem0, sem1):
    wid = lax.axis_index("c") * 16 + lax.axis_index("s")
    r0 = wid * ROWS_W
    i16 = lax.iota(jnp.int32, 16)

    pltpu.sync_copy(x_hbm, x_v)
    pltpu.sync_copy(ro_hbm.at[pl.ds(r0, ROWS_W + 8)],
                    ro_v.at[pl.ds(0, ROWS_W + 8)])
    s0 = ro_v[pl.ds(0, 16)][0]
    s1 = ro_v[pl.ds(ROWS_W, 16)][0]

    z16 = jnp.zeros((16,), jnp.float32)

    @pl.loop(0, ROWS_W + 16, step=16)
    def _(j):
        e_v[pl.ds(j, 16)] = z16

    k0 = s0 // CHUNK
    k1 = (s1 + CHUNK - 1) // CHUNK

    def start_chunk(k, sxb, ixb, sem):
        @pl.when(k < k1)
        def _():
            a = k * CHUNK
            pltpu.async_copy(sx_hbm.at[pl.ds(a, CHUNK)], sxb, sem)
            pltpu.async_copy(idx_hbm.at[pl.ds(a, CHUNK)], ixb, sem)

    def wait_chunk(do, sxb, ixb, sem):
        @pl.when(do)
        def _():
            pltpu.make_async_copy(sx_hbm.at[pl.ds(0, CHUNK)], sxb, sem).wait()
            pltpu.make_async_copy(idx_hbm.at[pl.ds(0, CHUNK)], ixb, sem).wait()

    def compute_chunk(k, gcur, carry, sxb, ixb):
        a = k * CHUNK

        @plsc.parallel_loop(0, NG, unroll=16)
        def _(j):
            o = j * 16
            iv = ixb[pl.ds(o, 16)]
            sv = sxb[pl.ds(o, 16)]
            g = plsc.load_gather(x_v, [iv])
            s_v[pl.ds(o, 16)] = plsc.cumsum(sv * g)

        @plsc.parallel_loop(0, NB, unroll=2)
        def _(b):
            tot = plsc.load_gather(s_v, [i16 * 16 + (b * 256 + 15)])
            l2 = plsc.cumsum(tot)
            l2_v[pl.ds(b * 16, 16)] = l2 - tot

        def p2b(bb, c2):
            t_ex = plsc.load_gather(l2_v, [i16 * 16 + (bb * 256 + 15)])
            t_g15 = plsc.load_gather(s_v, [i16 * 256 + (bb * 4096 + 255)])
            btot = t_ex + t_g15
            l3 = plsc.cumsum(btot) + c2
            l3_v[pl.ds(bb * 16, 16)] = l3 - btot
            return l3[15]

        carry = lax.fori_loop(0, NB // 16, p2b, carry)

        def prefix_at(gi):
            return (plsc.load_gather(s_v, [gi])
                    + plsc.load_gather(l2_v, [lax.shift_right_logical(gi, 4)])
                    + plsc.load_gather(l3_v, [lax.shift_right_logical(gi, 8)]))

        @pl.when(k == k0)
        def _():
            gb = (i16 * 0) + jnp.clip(s0 - 1 - a, 0, CHUNK - 1)
            pb = prefix_at(gb)
            e_v[pl.ds(0, 16)] = jnp.where((i16 == 15) & (s0 > a), pb, 0.0)

        cend = a + CHUNK

        def ext_cond(st2):
            g, stop = st2
            return jnp.logical_and(jnp.logical_not(stop), g < RG)

        def ext_body(st2):
            g, _ = st2
            tfirst = ro_v[pl.ds(g * 16, 16)][1]
            do = tfirst <= cend

            @pl.when(do)
            def _():
                t = plsc.load_gather(ro_v, [i16 + (1 + g * 16)])
                m = (t > a) & (t <= cend)
                gi = jnp.clip(t - 1 - a, 0, CHUNK - 1)
                pv = prefix_at(gi)
                eo = e_v[pl.ds(16 + g * 16, 16)]
                e_v[pl.ds(16 + g * 16, 16)] = jnp.where(m, pv, eo)

            return (jnp.where(do, g + 1, g), jnp.logical_not(do))

        gend, _ = lax.while_loop(ext_cond, ext_body,
                                 (gcur, jnp.asarray(False)))
        return jnp.maximum(gcur, gend - 1), carry

    def guarded_compute(k, gcur, carry, sxb, ixb):
        return lax.cond(
            k < k1,
            lambda: compute_chunk(k, gcur, carry, sxb, ixb),
            lambda: (gcur, carry),
        )

    start_chunk(k0, sx_v0, ix_v0, sem0)
    start_chunk(k0 + 1, sx_v1, ix_v1, sem1)

    def pair_body(i, st):
        gcur, carry = st
        ka = k0 + 2 * i
        kb = ka + 1
        wait_chunk(ka < k1, sx_v0, ix_v0, sem0)
        gcur, carry = guarded_compute(ka, gcur, carry, sx_v0, ix_v0)
        start_chunk(ka + 2, sx_v0, ix_v0, sem0)
        wait_chunk(kb < k1, sx_v1, ix_v1, sem1)
        gcur, carry = guarded_compute(kb, gcur, carry, sx_v1, ix_v1)
        start_chunk(kb + 2, sx_v1, ix_v1, sem1)
        return (gcur, carry)

    npairs = (k1 - k0 + 1) // 2
    lax.fori_loop(0, npairs, pair_body, (jnp.int32(0), jnp.float32(0.0)))

    @pl.loop(0, ROWS_W, step=16)
    def _(j):
        av = e_v[pl.ds(16 + j, 16)]
        bv = plsc.load_gather(e_v, [i16 + (15 + j)])
        y_v[pl.ds(j, 16)] = av - bv

    pltpu.sync_copy(y_v, o_hbm.at[pl.ds(r0, ROWS_W)])


def kernel(sx, x, y, selector_idx, row_end_offsets):
    del y
    ro_pad = jnp.concatenate(
        [row_end_offsets, jnp.broadcast_to(row_end_offsets[-1:], (7,))])
    return _spmv_sc(sx, x, selector_idx, ro_pad)
